# baseline (speedup 1.0000x reference)
import functools

import jax
import jax.numpy as jnp
from jax import lax
from jax.experimental import pallas as pl
from jax.experimental.pallas import tpu as pltpu
from jax.experimental.pallas import tpu_sc as plsc

_LANES = 16


@functools.cache
def _make_gather(R: int, S: int, D: int):
    scale = float(D) ** 0.5
    info = plsc.get_sparse_core_info()
    nw = info.num_cores * info.num_subcores
    r_per_w = R // nw
    NB = 2
    C = NB * S
    n_chunks = r_per_w // NB
    assert r_per_w % NB == 0 and R % nw == 0

    mesh = plsc.VectorSubcoreMesh(core_axis_name="c", subcore_axis_name="s")

    @functools.partial(
        pl.kernel,
        mesh=mesh,
        out_type=jax.ShapeDtypeStruct((R, S, D), jnp.float32),
        scratch_types=[
            pltpu.VMEM((C,), jnp.int32),
            pltpu.VMEM((C,), jnp.int32),
            pltpu.VMEM((C, 2 * D), jnp.float32),
            pltpu.VMEM((NB, S, D), jnp.float32),
            pltpu.SemaphoreType.DMA,
        ],
        compiler_params=pltpu.CompilerParams(use_tc_tiling_on_sc=True),
    )
    def gather_kernel(idx_hbm, pairs_hbm, out_hbm, idx_v, u_v, rows_v, out_v, sem):
        wid = lax.axis_index("s") * info.num_cores + lax.axis_index("c")
        row0 = wid * r_per_w

        def chunk_body(ci, carry):
            br = row0 + ci * NB
            pltpu.sync_copy(idx_hbm.at[pl.ds(br * S, C)], idx_v)

            def halve_body(k, c2):
                sl = pl.ds(k * _LANES, _LANES)
                u_v[sl] = lax.shift_right_logical(idx_v[sl], 1)
                return c2

            lax.fori_loop(0, C // _LANES, halve_body, 0)
            pltpu.async_copy(pairs_hbm.at[u_v], rows_v, sem).wait()

            def sel_body(g, c2):
                vvec = idx_v[pl.ds(g * _LANES, _LANES)]
                hvec = (vvec & 1) * D
                for ll in range(_LANES):
                    r = g * _LANES + ll
                    h = hvec[ll]
                    bi = jnp.where(r >= S, 1, 0)
                    si = r - bi * S
                    for j in range(D // _LANES):
                        out_v[bi, si, pl.ds(j * _LANES, _LANES)] = (
                            rows_v[r, pl.ds(h + j * _LANES, _LANES)] * scale
                        )
                return c2

            lax.fori_loop(0, C // _LANES, sel_body, 0)
            pltpu.sync_copy(out_v, out_hbm.at[pl.ds(br, NB)])
            return carry

        lax.fori_loop(0, n_chunks, chunk_body, 0)

    return gather_kernel


def kernel(x, table):
    R, S = x.shape
    V, D = table.shape
    xf = x.reshape(R * S)
    pairs = table.reshape(V // 2, 2 * D)
    return _make_gather(R, S, D)(xf, pairs)
